# Initial kernel scaffold; baseline (speedup 1.0000x reference)
import jax
import jax.numpy as jnp
from jax.experimental import pallas as pl


def kernel(x, batch):
    raise NotImplementedError("write your pallas kernel here")


# speedup vs baseline: 1.5543x; 1.5543x over previous
import jax
import jax.numpy as jnp
from jax import lax
from jax.experimental import pallas as pl
from jax.experimental.pallas import tpu as pltpu
from jax.experimental.pallas import tpu_sc as plsc

N = 160000
D = 256
S = 1024
NC = 2
NS = 16
NW = NC * NS
SEG_PER_W = S // NW
L = 16
DV = D // L
OD = 3 * D
C = 128
IDC = 10000
NIDC = N // IDC


def _body(x_hbm, b_hbm, out_hbm, ids_a, idbuf0, idbuf1, xbuf0, xbuf1,
          outbuf, sem0, sem1):
  wid = lax.axis_index("s") * NC + lax.axis_index("c")
  lo = wid * SEG_PER_W
  hi = lo + SEG_PER_W

  zero = jnp.zeros((L,), jnp.float32)

  def zvec(t, _):
    outbuf[pl.ds(t * L, L)] = zero
    return 0

  lax.fori_loop(0, SEG_PER_W * OD // L, zvec, 0)

  def cnt_chunk(t, carry):
    clo, chi = carry
    pltpu.sync_copy(b_hbm.at[pl.ds(t * IDC, IDC)], ids_a)

    def cnt_vec(v, carry):
      clo, chi = carry
      vec = ids_a[pl.ds(v * L, L)]
      one = jnp.ones((L,), jnp.int32)
      z = jnp.zeros((L,), jnp.int32)
      clo = clo + jnp.where(vec < lo, one, z)
      chi = chi + jnp.where(vec < hi, one, z)
      return clo, chi

    return lax.fori_loop(0, IDC // L, cnt_vec, (clo, chi), unroll=4)

  clo, chi = lax.fori_loop(
      0, NIDC, cnt_chunk,
      (jnp.zeros((L,), jnp.int32), jnp.zeros((L,), jnp.int32)))

  def _hsum(v):
    s = v[0]
    for i in range(1, L):
      s = s + v[i]
    return s

  start = _hsum(clo)
  end = _hsum(chi)

  j_lo = start >> 7
  j_hi = (end + C - 1) >> 7
  npair = (j_hi - j_lo + 1) >> 1

  def flush(cur, cnt, sums, maxs):
    base = (cur - lo) * OD
    cntf = cnt.astype(jnp.float32)
    for k in range(DV):
      outbuf[pl.ds(base + k * L, L)] = sums[k] / cntf
      outbuf[pl.ds(base + D + k * L, L)] = maxs[k]
      outbuf[pl.ds(base + 2 * D + k * L, L)] = sums[k]

  def dma_descs(j, xbuf, idbuf, sem):
    cp_x = pltpu.make_async_copy(x_hbm.at[pl.ds(j * C * D, C * D)], xbuf, sem)
    cp_i = pltpu.make_async_copy(b_hbm.at[pl.ds(j * C, C)], idbuf, sem)
    return cp_x, cp_i

  def dma_start(j, xbuf, idbuf, sem):
    cp_x, cp_i = dma_descs(j, xbuf, idbuf, sem)
    cp_x.start()
    cp_i.start()

  def dma_wait(j, xbuf, idbuf, sem):
    cp_x, cp_i = dma_descs(j, xbuf, idbuf, sem)
    cp_x.wait()
    cp_i.wait()

  def process_chunk(xbuf, idbuf, carry):
    def grp_body(g, carry):
      idvec = idbuf[pl.ds(g * L, L)]

      def row_step(lane, carry):
        cur, cnt, sums, maxs = carry
        sid = idvec[lane]
        inside = jnp.logical_and(sid >= lo, sid < hi)
        is_new = jnp.logical_and(inside, sid != cur)

        @pl.when(jnp.logical_and(is_new, cur >= 0))
        def _():
          flush(cur, cnt, sums, maxs)

        rbase = (g * L + lane) * D
        row = [xbuf[pl.ds(rbase + k * L, L)] for k in range(DV)]
        acc = jnp.logical_and(inside, jnp.logical_not(is_new))
        new_sums = [
            jnp.where(is_new, row[k],
                      jnp.where(acc, sums[k] + row[k], sums[k]))
            for k in range(DV)
        ]
        new_maxs = [
            jnp.where(is_new, row[k],
                      jnp.where(acc, jnp.maximum(maxs[k], row[k]), maxs[k]))
            for k in range(DV)
        ]
        cnt = jnp.where(is_new, 1, cnt + inside.astype(jnp.int32))
        cur = jnp.where(is_new, sid, cur)
        return cur, cnt, new_sums, new_maxs

      for lane in range(L):
        carry = row_step(lane, carry)
      return carry

    return lax.fori_loop(0, C // L, grp_body, carry)

  @pl.when(j_lo < j_hi)
  def _():
    dma_start(j_lo, xbuf0, idbuf0, sem0)

  @pl.when(j_lo + 1 < j_hi)
  def _():
    dma_start(j_lo + 1, xbuf1, idbuf1, sem1)

  def pair_body(k, carry):
    j = j_lo + 2 * k
    dma_wait(j, xbuf0, idbuf0, sem0)
    carry = process_chunk(xbuf0, idbuf0, carry)

    @pl.when(j + 2 < j_hi)
    def _():
      dma_start(j + 2, xbuf0, idbuf0, sem0)

    @pl.when(j + 1 < j_hi)
    def _():
      dma_wait(j + 1, xbuf1, idbuf1, sem1)

    @pl.when(j + 1 >= j_hi)
    def _():
      sent = jnp.full((L,), -1, jnp.int32)

      def fill(v, _):
        idbuf1[pl.ds(v * L, L)] = sent
        return 0

      lax.fori_loop(0, C // L, fill, 0)

    carry = process_chunk(xbuf1, idbuf1, carry)

    @pl.when(j + 3 < j_hi)
    def _():
      dma_start(j + 3, xbuf1, idbuf1, sem1)

    return carry

  zf = jnp.zeros((L,), jnp.float32)
  carry0 = (jnp.int32(-1), jnp.int32(0), [zf] * DV, [zf] * DV)
  cur, cnt, sums, maxs = lax.fori_loop(0, npair, pair_body, carry0)

  @pl.when(cur >= 0)
  def _():
    flush(cur, cnt, sums, maxs)

  pltpu.sync_copy(outbuf, out_hbm.at[pl.ds(lo * OD, SEG_PER_W * OD)])


@jax.jit
def _pool(x, batch):
  mesh = plsc.VectorSubcoreMesh(
      core_axis_name="c", subcore_axis_name="s", num_cores=NC, num_subcores=NS)
  out_flat = pl.kernel(
      _body,
      out_type=jax.ShapeDtypeStruct((S * OD,), jnp.float32),
      mesh=mesh,
      scratch_types=[
          pltpu.VMEM((IDC,), jnp.int32),
          pltpu.VMEM((C,), jnp.int32),
          pltpu.VMEM((C,), jnp.int32),
          pltpu.VMEM((C * D,), jnp.float32),
          pltpu.VMEM((C * D,), jnp.float32),
          pltpu.VMEM((SEG_PER_W * OD,), jnp.float32),
          pltpu.SemaphoreType.DMA,
          pltpu.SemaphoreType.DMA,
      ],
  )(x.reshape(N * D), batch)
  return out_flat.reshape(S, OD)


def kernel(x, batch):
  return _pool(x, batch.astype(jnp.int32))
